# Initial kernel scaffold; baseline (speedup 1.0000x reference)
import functools

import jax
import jax.numpy as jnp
from jax import lax
from jax.experimental import pallas as pl
from jax.experimental.pallas import tpu as pltpu
from jax.experimental.pallas import tpu_sc as plsc

NP = 10240
DH = 128
NG = 64
NCLS = 16
BLK = 256
NWORK = 32
ECH = 128
ROWS_PER_SUB = NP // 16
ZCH = ROWS_PER_SUB // ECH


def _sc_mesh():
    return plsc.VectorSubcoreMesh(core_axis_name="c", subcore_axis_name="s")


def _make_deg_kernel(nch):
    @functools.partial(
        pl.kernel,
        out_type=jax.ShapeDtypeStruct((2, NP, 16), jnp.float32),
        mesh=_sc_mesh(),
        scratch_types=[
            pltpu.VMEM_SHARED((NP, 16), jnp.float32),
            pltpu.VMEM((ECH,), jnp.int32),
            pltpu.VMEM((ECH, 16), jnp.float32),
            pltpu.VMEM((ECH, 16), jnp.float32),
        ],
    )
    def deg_k(dst3, out, acc, dst_v, ones_v, buf_v):
        c = lax.axis_index("c")
        s = lax.axis_index("s")
        wid = s * 2 + c

        def init(i, _):
            ones_v[i, :] = jnp.ones((16,), jnp.float32)
            buf_v[i, :] = jnp.zeros((16,), jnp.float32)
            return 0

        lax.fori_loop(0, ECH, init, 0)

        def zacc(i, _):
            pltpu.sync_copy(buf_v, acc.at[pl.ds(s * ROWS_PER_SUB + i * ECH, ECH)])
            return 0

        lax.fori_loop(0, ZCH, zacc, 0)
        plsc.subcore_barrier()

        def chunk(j, _):
            pltpu.sync_copy(dst3.at[wid, j], dst_v)
            pltpu.sync_copy(ones_v, acc.at[dst_v], add=True)
            return 0

        lax.fori_loop(0, nch, chunk, 0)
        plsc.subcore_barrier()

        def wb(i, _):
            r0 = s * ROWS_PER_SUB + i * ECH
            pltpu.sync_copy(acc.at[pl.ds(r0, ECH)], buf_v)
            pltpu.sync_copy(buf_v, out.at[c, pl.ds(r0, ECH)])
            return 0

        lax.fori_loop(0, ZCH, wb, 0)

    return deg_k


def _make_scatter_kernel(nch):
    @functools.partial(
        pl.kernel,
        out_type=jax.ShapeDtypeStruct((2, NP, DH), jnp.float32),
        mesh=_sc_mesh(),
        scratch_types=[
            pltpu.VMEM_SHARED((NP, DH), jnp.float32),
            pltpu.VMEM((ECH,), jnp.int32),
            pltpu.VMEM((ECH,), jnp.int32),
            pltpu.VMEM((ECH, DH), jnp.float32),
            pltpu.SemaphoreType.DMA,
        ],
    )
    def scatter_k(u_hbm, src3, dst3, out, acc, src_v, dst_v, rows_v, sem):
        c = lax.axis_index("c")
        s = lax.axis_index("s")
        wid = s * 2 + c

        def zrow(i, _):
            for jj in range(DH // 16):
                rows_v[i, pl.ds(jj * 16, 16)] = jnp.zeros((16,), jnp.float32)
            return 0

        lax.fori_loop(0, ECH, zrow, 0)

        def zacc(i, _):
            pltpu.sync_copy(rows_v, acc.at[pl.ds(s * ROWS_PER_SUB + i * ECH, ECH)])
            return 0

        lax.fori_loop(0, ZCH, zacc, 0)
        plsc.subcore_barrier()

        def chunk(j, _):
            pltpu.sync_copy(src3.at[wid, j], src_v)
            pltpu.sync_copy(dst3.at[wid, j], dst_v)
            pltpu.async_copy(u_hbm.at[src_v], rows_v, sem).wait()
            pltpu.sync_copy(rows_v, acc.at[dst_v], add=True)
            return 0

        lax.fori_loop(0, nch, chunk, 0)
        plsc.subcore_barrier()

        def wb(i, _):
            r0 = s * ROWS_PER_SUB + i * ECH
            pltpu.sync_copy(acc.at[pl.ds(r0, ECH)], rows_v)
            pltpu.sync_copy(rows_v, out.at[c, pl.ds(r0, ECH)])
            return 0

        lax.fori_loop(0, ZCH, wb, 0)

    return scatter_k


def _prep_body(degp_ref, x_ref, w_ref, u_ref, dis_ref):
    deg = degp_ref[0, :, 0:1] + degp_ref[1, :, 0:1] + 1.0
    dis = lax.rsqrt(deg)
    u_ref[...] = dis * jnp.dot(x_ref[...], w_ref[...],
                               preferred_element_type=jnp.float32)
    dis_ref[...] = dis


def _layer_body(p_ref, u_ref, dis_ref, b_ref, w_ref, out_ref):
    dis = dis_ref[...]
    h = dis * (p_ref[0] + p_ref[1] + u_ref[...]) + b_ref[...]
    h = jnp.maximum(h, 0.0)
    out_ref[...] = dis * jnp.dot(h, w_ref[...],
                                 preferred_element_type=jnp.float32)


def _final_body(nblk, p_ref, u_ref, dis_ref, b_ref, batch_ref, wlin_ref,
                blin_ref, out_ref, sums_ref, cnt_ref):
    j = pl.program_id(0)

    @pl.when(j == 0)
    def _():
        sums_ref[...] = jnp.zeros((NG, DH), jnp.float32)
        cnt_ref[...] = jnp.zeros((NG, DH), jnp.float32)

    h4 = dis_ref[...] * (p_ref[0] + p_ref[1] + u_ref[...]) + b_ref[...]
    gids = lax.broadcasted_iota(jnp.int32, (BLK, NG), 1)
    onehot = (batch_ref[...] == gids).astype(jnp.float32)
    dn = (((0,), (0,)), ((), ()))
    sums_ref[...] += lax.dot_general(onehot, h4, dn,
                                     preferred_element_type=jnp.float32)
    cnt_ref[...] += lax.dot_general(onehot, jnp.ones((BLK, DH), jnp.float32),
                                    dn, preferred_element_type=jnp.float32)

    @pl.when(j == nblk - 1)
    def _():
        pooled = sums_ref[...] / jnp.maximum(cnt_ref[...], 1.0)
        out_ref[...] = jnp.dot(pooled, wlin_ref[...],
                               preferred_element_type=jnp.float32) + blin_ref[...]


def kernel(x, edge_index, batch, W1, b1, W2, b2, W3, b3, W4, b4, Wlin, blin):
    f32 = jnp.float32
    n = x.shape[0]
    e = edge_index.shape[1]
    src = edge_index[0].astype(jnp.int32)
    dst = edge_index[1].astype(jnp.int32)

    nch = -(-e // (NWORK * ECH))
    ep = NWORK * nch * ECH
    pad_e = ep - e
    src3 = jnp.concatenate([src, jnp.full((pad_e,), n, jnp.int32)]).reshape(
        NWORK, nch, ECH)
    dst3 = jnp.concatenate([dst, jnp.full((pad_e,), n, jnp.int32)]).reshape(
        NWORK, nch, ECH)

    x_pad = jnp.pad(x.astype(f32), ((0, NP - n), (0, 0)))
    batch_p = jnp.pad(batch.astype(jnp.int32), (0, NP - n),
                      constant_values=NG).reshape(NP, 1)
    b1r, b2r, b3r, b4r = (v.reshape(1, DH) for v in (b1, b2, b3, b4))
    wlin_p = jnp.pad(Wlin.astype(f32), ((0, 0), (0, DH - NCLS)))
    blin_p = jnp.pad(blin.astype(f32), (0, DH - NCLS)).reshape(1, DH)

    nblk = NP // BLK
    deg_k = _make_deg_kernel(nch)
    scat_k = _make_scatter_kernel(nch)

    degp = deg_k(dst3)

    row_spec = pl.BlockSpec((BLK, DH), lambda i: (i, 0))
    col1_spec = pl.BlockSpec((BLK, 1), lambda i: (i, 0))
    pair_spec = pl.BlockSpec((2, BLK, DH), lambda i: (0, i, 0))
    w_spec = pl.BlockSpec((DH, DH), lambda i: (0, 0))
    b_spec = pl.BlockSpec((1, DH), lambda i: (0, 0))

    u1, dis = pl.pallas_call(
        _prep_body,
        grid=(nblk,),
        in_specs=[pl.BlockSpec((2, BLK, 16), lambda i: (0, i, 0)),
                  row_spec, w_spec],
        out_specs=[row_spec, col1_spec],
        out_shape=[jax.ShapeDtypeStruct((NP, DH), f32),
                   jax.ShapeDtypeStruct((NP, 1), f32)],
    )(degp, x_pad, W1)

    layer = pl.pallas_call(
        _layer_body,
        grid=(nblk,),
        in_specs=[pair_spec, row_spec, col1_spec, b_spec, w_spec],
        out_specs=row_spec,
        out_shape=jax.ShapeDtypeStruct((NP, DH), f32),
    )

    p = scat_k(u1, src3, dst3)
    u2 = layer(p, u1, dis, b1r, W2)
    p = scat_k(u2, src3, dst3)
    u3 = layer(p, u2, dis, b2r, W3)
    p = scat_k(u3, src3, dst3)
    u4 = layer(p, u3, dis, b3r, W4)
    p = scat_k(u4, src3, dst3)

    res = pl.pallas_call(
        functools.partial(_final_body, nblk),
        grid=(nblk,),
        in_specs=[pair_spec, row_spec, col1_spec, b_spec,
                  pl.BlockSpec((BLK, 1), lambda i: (i, 0)),
                  w_spec, b_spec],
        out_specs=pl.BlockSpec((NG, DH), lambda i: (0, 0)),
        out_shape=jax.ShapeDtypeStruct((NG, DH), f32),
        scratch_shapes=[pltpu.VMEM((NG, DH), f32), pltpu.VMEM((NG, DH), f32)],
    )(p, u4, dis, b4r, batch_p, wlin_p, blin_p)

    return res[:, :NCLS]


# speedup vs baseline: 8.0516x; 8.0516x over previous
import functools

import jax
import jax.numpy as jnp
from jax import lax
from jax.experimental import pallas as pl
from jax.experimental.pallas import tpu as pltpu
from jax.experimental.pallas import tpu_sc as plsc

NP = 10240
DH = 128
NG = 64
NCLS = 16
BLK = 256
NWORK = 32
ECH = 128
ROWS_PER_SUB = NP // 16
ZCH = ROWS_PER_SUB // ECH


def _sc_mesh():
    return plsc.VectorSubcoreMesh(core_axis_name="c", subcore_axis_name="s")


def _make_deg_kernel(nch):
    @functools.partial(
        pl.kernel,
        out_type=jax.ShapeDtypeStruct((2, NP, DH), jnp.float32),
        mesh=_sc_mesh(),
        scratch_types=[
            pltpu.VMEM_SHARED((NP, DH), jnp.float32),
            pltpu.VMEM((ECH,), jnp.int32),
            pltpu.VMEM((ECH, DH), jnp.float32),
            pltpu.VMEM((ECH, DH), jnp.float32),
        ],
    )
    def deg_k(dst3, out, acc, dst_v, ones_v, buf_v):
        c = lax.axis_index("c")
        s = lax.axis_index("s")
        wid = s * 2 + c

        def init(i, _):
            for jj in range(DH // 16):
                ones_v[i, pl.ds(jj * 16, 16)] = jnp.ones((16,), jnp.float32)
                buf_v[i, pl.ds(jj * 16, 16)] = jnp.zeros((16,), jnp.float32)
            return 0

        lax.fori_loop(0, ECH, init, 0)

        def zacc(i, _):
            pltpu.sync_copy(buf_v, acc.at[pl.ds(s * ROWS_PER_SUB + i * ECH, ECH)])
            return 0

        lax.fori_loop(0, ZCH, zacc, 0)
        plsc.subcore_barrier()

        def chunk(j, _):
            pltpu.sync_copy(dst3.at[wid, j], dst_v)
            pltpu.sync_copy(ones_v, acc.at[dst_v], add=True)
            return 0

        lax.fori_loop(0, nch, chunk, 0)
        plsc.subcore_barrier()

        def wb(i, _):
            r0 = s * ROWS_PER_SUB + i * ECH
            pltpu.sync_copy(acc.at[pl.ds(r0, ECH)], buf_v)
            pltpu.sync_copy(buf_v, out.at[c, pl.ds(r0, ECH)])
            return 0

        lax.fori_loop(0, ZCH, wb, 0)

    return deg_k


def _make_scatter_kernel(nch):
    @functools.partial(
        pl.kernel,
        out_type=jax.ShapeDtypeStruct((2, NP, DH), jnp.float32),
        mesh=_sc_mesh(),
        scratch_types=[
            pltpu.VMEM_SHARED((NP, DH), jnp.float32),
            pltpu.VMEM((ECH,), jnp.int32),
            pltpu.VMEM((ECH,), jnp.int32),
            pltpu.VMEM((ECH, DH), jnp.float32),
            pltpu.SemaphoreType.DMA,
        ],
    )
    def scatter_k(u_hbm, src3, dst3, out, acc, src_v, dst_v, rows_v, sem):
        c = lax.axis_index("c")
        s = lax.axis_index("s")
        wid = s * 2 + c

        def zrow(i, _):
            for jj in range(DH // 16):
                rows_v[i, pl.ds(jj * 16, 16)] = jnp.zeros((16,), jnp.float32)
            return 0

        lax.fori_loop(0, ECH, zrow, 0)

        def zacc(i, _):
            pltpu.sync_copy(rows_v, acc.at[pl.ds(s * ROWS_PER_SUB + i * ECH, ECH)])
            return 0

        lax.fori_loop(0, ZCH, zacc, 0)
        plsc.subcore_barrier()

        def chunk(j, _):
            pltpu.sync_copy(src3.at[wid, j], src_v)
            pltpu.sync_copy(dst3.at[wid, j], dst_v)
            pltpu.async_copy(u_hbm.at[src_v], rows_v, sem).wait()
            pltpu.sync_copy(rows_v, acc.at[dst_v], add=True)
            return 0

        lax.fori_loop(0, nch, chunk, 0)
        plsc.subcore_barrier()

        def wb(i, _):
            r0 = s * ROWS_PER_SUB + i * ECH
            pltpu.sync_copy(acc.at[pl.ds(r0, ECH)], rows_v)
            pltpu.sync_copy(rows_v, out.at[c, pl.ds(r0, ECH)])
            return 0

        lax.fori_loop(0, ZCH, wb, 0)

    return scatter_k


def _prep_body(degp_ref, x_ref, w_ref, u_ref, dis_ref):
    deg = degp_ref[0, :, 0:1] + degp_ref[1, :, 0:1] + 1.0
    dis = lax.rsqrt(deg)
    u_ref[...] = dis * jnp.dot(x_ref[...], w_ref[...],
                               preferred_element_type=jnp.float32)
    dis_ref[...] = dis


def _layer_body(p_ref, u_ref, dis_ref, b_ref, w_ref, out_ref):
    dis = dis_ref[...]
    h = dis * (p_ref[0] + p_ref[1] + u_ref[...]) + b_ref[...]
    h = jnp.maximum(h, 0.0)
    out_ref[...] = dis * jnp.dot(h, w_ref[...],
                                 preferred_element_type=jnp.float32)


def _final_body(nblk, p_ref, u_ref, dis_ref, b_ref, batch_ref, wlin_ref,
                blin_ref, out_ref, sums_ref, cnt_ref):
    j = pl.program_id(0)

    @pl.when(j == 0)
    def _():
        sums_ref[...] = jnp.zeros((NG, DH), jnp.float32)
        cnt_ref[...] = jnp.zeros((NG, DH), jnp.float32)

    h4 = dis_ref[...] * (p_ref[0] + p_ref[1] + u_ref[...]) + b_ref[...]
    gids = lax.broadcasted_iota(jnp.int32, (BLK, NG), 1)
    onehot = (batch_ref[...] == gids).astype(jnp.float32)
    dn = (((0,), (0,)), ((), ()))
    sums_ref[...] += lax.dot_general(onehot, h4, dn,
                                     preferred_element_type=jnp.float32)
    cnt_ref[...] += lax.dot_general(onehot, jnp.ones((BLK, DH), jnp.float32),
                                    dn, preferred_element_type=jnp.float32)

    @pl.when(j == nblk - 1)
    def _():
        pooled = sums_ref[...] / jnp.maximum(cnt_ref[...], 1.0)
        out_ref[...] = jnp.dot(pooled, wlin_ref[...],
                               preferred_element_type=jnp.float32) + blin_ref[...]


def kernel(x, edge_index, batch, W1, b1, W2, b2, W3, b3, W4, b4, Wlin, blin):
    f32 = jnp.float32
    n = x.shape[0]
    e = edge_index.shape[1]
    src = edge_index[0].astype(jnp.int32)
    dst = edge_index[1].astype(jnp.int32)

    nch = -(-e // (NWORK * ECH))
    ep = NWORK * nch * ECH
    pad_e = ep - e
    src3 = jnp.concatenate([src, jnp.full((pad_e,), n, jnp.int32)]).reshape(
        NWORK, nch, ECH)
    dst3 = jnp.concatenate([dst, jnp.full((pad_e,), n, jnp.int32)]).reshape(
        NWORK, nch, ECH)

    x_pad = jnp.pad(x.astype(f32), ((0, NP - n), (0, 0)))
    batch_p = jnp.pad(batch.astype(jnp.int32), (0, NP - n),
                      constant_values=NG).reshape(NP, 1)
    b1r, b2r, b3r, b4r = (v.reshape(1, DH) for v in (b1, b2, b3, b4))
    wlin_p = jnp.pad(Wlin.astype(f32), ((0, 0), (0, DH - NCLS)))
    blin_p = jnp.pad(blin.astype(f32), (0, DH - NCLS)).reshape(1, DH)

    nblk = NP // BLK
    deg_k = _make_deg_kernel(nch)
    scat_k = _make_scatter_kernel(nch)

    degp = deg_k(dst3)

    row_spec = pl.BlockSpec((BLK, DH), lambda i: (i, 0))
    col1_spec = pl.BlockSpec((BLK, 1), lambda i: (i, 0))
    pair_spec = pl.BlockSpec((2, BLK, DH), lambda i: (0, i, 0))
    w_spec = pl.BlockSpec((DH, DH), lambda i: (0, 0))
    b_spec = pl.BlockSpec((1, DH), lambda i: (0, 0))

    u1, dis = pl.pallas_call(
        _prep_body,
        grid=(nblk,),
        in_specs=[pl.BlockSpec((2, BLK, DH), lambda i: (0, i, 0)),
                  row_spec, w_spec],
        out_specs=[row_spec, col1_spec],
        out_shape=[jax.ShapeDtypeStruct((NP, DH), f32),
                   jax.ShapeDtypeStruct((NP, 1), f32)],
    )(degp, x_pad, W1)

    layer = pl.pallas_call(
        _layer_body,
        grid=(nblk,),
        in_specs=[pair_spec, row_spec, col1_spec, b_spec, w_spec],
        out_specs=row_spec,
        out_shape=jax.ShapeDtypeStruct((NP, DH), f32),
    )

    p = scat_k(u1, src3, dst3)
    u2 = layer(p, u1, dis, b1r, W2)
    p = scat_k(u2, src3, dst3)
    u3 = layer(p, u2, dis, b2r, W3)
    p = scat_k(u3, src3, dst3)
    u4 = layer(p, u3, dis, b3r, W4)
    p = scat_k(u4, src3, dst3)

    res = pl.pallas_call(
        functools.partial(_final_body, nblk),
        grid=(nblk,),
        in_specs=[pair_spec, row_spec, col1_spec, b_spec,
                  pl.BlockSpec((BLK, 1), lambda i: (i, 0)),
                  w_spec, b_spec],
        out_specs=pl.BlockSpec((NG, DH), lambda i: (0, 0)),
        out_shape=jax.ShapeDtypeStruct((NG, DH), f32),
        scratch_shapes=[pltpu.VMEM((NG, DH), f32), pltpu.VMEM((NG, DH), f32)],
    )(p, u4, dis, b4r, batch_p, wlin_p, blin_p)

    return res[:, :NCLS]
